# Initial kernel scaffold; baseline (speedup 1.0000x reference)
import jax
import jax.numpy as jnp
from jax.experimental import pallas as pl


def kernel(x, emb):
    raise NotImplementedError("write your pallas kernel here")


# speedup vs baseline: 1.6773x; 1.6773x over previous
import jax
import jax.numpy as jnp
from jax.experimental import pallas as pl
from jax.experimental.pallas import tpu as pltpu

S_BLK = 512


def _add_body(x_ref, emb_ref, out_ref):
    out_ref[...] = x_ref[...] + emb_ref[...]


def kernel(x, emb):
    B, T, D = x.shape
    grid = (T // S_BLK, B)
    return pl.pallas_call(
        _add_body,
        grid=grid,
        in_specs=[
            pl.BlockSpec((1, S_BLK, D), lambda i, b: (b, i, 0)),
            pl.BlockSpec((S_BLK, D), lambda i, b: (i, 0)),
        ],
        out_specs=pl.BlockSpec((1, S_BLK, D), lambda i, b: (b, i, 0)),
        out_shape=jax.ShapeDtypeStruct((B, T, D), x.dtype),
    )(x, emb)


# speedup vs baseline: 1.8585x; 1.1080x over previous
import jax
import jax.numpy as jnp
from jax.experimental import pallas as pl
from jax.experimental.pallas import tpu as pltpu

S_BLK = 1024


def _add_body(x_ref, emb_ref, out_ref):
    out_ref[...] = x_ref[...] + emb_ref[...]


def kernel(x, emb):
    B, T, D = x.shape
    grid = (T // S_BLK, B)
    return pl.pallas_call(
        _add_body,
        grid=grid,
        in_specs=[
            pl.BlockSpec((1, S_BLK, D), lambda i, b: (b, i, 0)),
            pl.BlockSpec((S_BLK, D), lambda i, b: (i, 0)),
        ],
        out_specs=pl.BlockSpec((1, S_BLK, D), lambda i, b: (b, i, 0)),
        out_shape=jax.ShapeDtypeStruct((B, T, D), x.dtype),
    )(x, emb)


# speedup vs baseline: 1.9730x; 1.0616x over previous
import jax
import jax.numpy as jnp
from jax.experimental import pallas as pl
from jax.experimental.pallas import tpu as pltpu

S_BLK = 2048


def _add_body(x_ref, emb_ref, out_ref):
    out_ref[...] = x_ref[...] + emb_ref[...]


def kernel(x, emb):
    B, T, D = x.shape
    grid = (T // S_BLK, B)
    return pl.pallas_call(
        _add_body,
        grid=grid,
        in_specs=[
            pl.BlockSpec((1, S_BLK, D), lambda i, b: (b, i, 0)),
            pl.BlockSpec((S_BLK, D), lambda i, b: (i, 0)),
        ],
        out_specs=pl.BlockSpec((1, S_BLK, D), lambda i, b: (b, i, 0)),
        out_shape=jax.ShapeDtypeStruct((B, T, D), x.dtype),
    )(x, emb)
